# baseline (speedup 1.0000x reference)
import jax
import jax.numpy as jnp
import numpy as np
from jax.experimental import pallas as pl

_N = 10000
_G = 256


def _conv(x, src, dst, Wq, bq, Wk, bk, Wv, bv, Ws, bs, num_nodes):
    q = x @ Wq + bq
    k = x @ Wk + bk
    v = x @ Wv + bv
    d = q.shape[-1]
    logits = jnp.sum(q[dst] * k[src], axis=-1) / jnp.sqrt(float(d))
    m = jax.ops.segment_max(logits, dst, num_segments=num_nodes)
    m = jnp.where(jnp.isfinite(m), m, 0.0)
    ex = jnp.exp(logits - m[dst])
    denom = jax.ops.segment_sum(ex, dst, num_segments=num_nodes)
    alpha = ex / (denom[dst] + 1e-16)
    agg = jax.ops.segment_sum(alpha[:, None] * v[src], dst, num_segments=num_nodes)
    return agg + (x @ Ws + bs)


def _copy_kernel(x_ref, o_ref):
    o_ref[...] = x_ref[...]


def kernel(x, edge_index, edge_attr, batch, Wq1, bq1, Wk1, bk1, Wv1, bv1, Ws1, bs1,
           Wq2, bq2, Wk2, bk2, Wv2, bv2, Ws2, bs2):
    src = edge_index[0]
    dst = edge_index[1]
    h = _conv(x.astype(jnp.float32), src, dst, Wq1, bq1, Wk1, bk1, Wv1, bv1, Ws1, bs1, _N)
    h = jax.nn.relu(h)
    h = _conv(h, src, dst, Wq2, bq2, Wk2, bk2, Wv2, bv2, Ws2, bs2, _N)
    sums = jax.ops.segment_sum(h, batch, num_segments=_G)
    counts = jax.ops.segment_sum(jnp.ones((_N,), jnp.float32), batch, num_segments=_G)
    out = sums / jnp.maximum(counts, 1.0)[:, None]
    return pl.pallas_call(
        _copy_kernel,
        out_shape=jax.ShapeDtypeStruct(out.shape, out.dtype),
    )(out)


# speedup vs baseline: 7.6184x; 7.6184x over previous
import functools

import jax
import jax.numpy as jnp
from jax import lax
from jax.experimental import pallas as pl
from jax.experimental.pallas import tpu as pltpu
from jax.experimental.pallas import tpu_sc as plsc

_N = 10000
_E = 320000
_D = 128
_G = 256

_NC = 2
_NS = 16
_NW = _NC * _NS
_L = 16

_NPAD = 10240
_SLICE = _NPAD // _NS
_EW = _E // _NW
_B = 80
_NBLK = _EW // _B
_RS = _N // _NS

_NEG = -3.0e38
_SCALE = 1.0 / (128.0 ** 0.5)

_mesh = plsc.VectorSubcoreMesh(core_axis_name="c", subcore_axis_name="s",
                               num_cores=_NC, num_subcores=_NS)


def _take(x, idx):
    return jnp.take_along_axis(x, idx, axis=0)


@functools.partial(
    pl.kernel,
    out_type=[jax.ShapeDtypeStruct((_E,), jnp.float32),
              jax.ShapeDtypeStruct((_NC, _NPAD), jnp.float32)],
    mesh=_mesh,
    compiler_params=pltpu.CompilerParams(needs_layout_passes=False),
    scratch_types=[
        pltpu.VMEM((_B,), jnp.int32),
        pltpu.VMEM((_B,), jnp.int32),
        pltpu.VMEM((_B, _D), jnp.float32),
        pltpu.VMEM((_B, _D), jnp.float32),
        pltpu.VMEM((_B,), jnp.float32),
        pltpu.VMEM((_L * _L,), jnp.float32),
        pltpu.VMEM((_NPAD,), jnp.float32),
        pltpu.VMEM((_SLICE,), jnp.float32),
        pltpu.VMEM((_SLICE,), jnp.float32),
        pltpu.VMEM_SHARED((_NS, _NPAD), jnp.float32),
        pltpu.SemaphoreType.DMA,
        pltpu.SemaphoreType.DMA,
    ],
)
def _sc_logits_max(q_hbm, k_hbm, src_hbm, dst_hbm, logits_out, msc_out,
                   srcidx, dstidx, qrows, krows, lblk, accm, mpriv,
                   redacc, redbuf, shared_m, semq, semk):
    cid = lax.axis_index("c")
    sid = lax.axis_index("s")
    wid = sid * _NC + cid
    base = wid * _EW
    iota = lax.iota(jnp.int32, _L)
    neg = jnp.full((_L,), _NEG, jnp.float32)

    def initbody(i, _):
        mpriv[pl.ds(i * _L, _L)] = neg
        return 0
    lax.fori_loop(0, _NPAD // _L, initbody, 0)

    def block(blk, _):
        off = base + blk * _B
        pltpu.sync_copy(src_hbm.at[pl.ds(off, _B)], srcidx)
        pltpu.sync_copy(dst_hbm.at[pl.ds(off, _B)], dstidx)
        cq = pltpu.async_copy(q_hbm.at[dstidx], qrows, semq)
        ck = pltpu.async_copy(k_hbm.at[srcidx], krows, semk)
        cq.wait()
        ck.wait()

        def group(g, _):
            gb = g * _L
            for r in range(_L):
                e = gb + r
                acc = qrows[e, pl.ds(0, _L)] * krows[e, pl.ds(0, _L)]
                for c in range(1, _D // _L):
                    acc = acc + (qrows[e, pl.ds(c * _L, _L)] *
                                 krows[e, pl.ds(c * _L, _L)])
                accm[pl.ds(r * _L, _L)] = acc
            tot = plsc.load_gather(accm, [iota * _L])
            for l in range(1, _L):
                tot = tot + plsc.load_gather(accm, [iota * _L + l])
            lv = tot * _SCALE
            lblk[pl.ds(gb, _L)] = lv

            dstv = dstidx[pl.ds(gb, _L)]
            sk, sv = plsc.sort_key_val(dstv, lv)
            for s in (1, 2, 4, 8):
                kprev = _take(sk, jnp.maximum(iota - s, 0))
                vprev = _take(sv, jnp.maximum(iota - s, 0))
                same = (kprev == sk) & (iota >= s)
                sv = jnp.where(same, jnp.maximum(sv, vprev), sv)
            nxt = _take(sk, jnp.minimum(iota + 1, _L - 1))
            last = (sk != nxt) | (iota == _L - 1)
            cur = plsc.load_gather(mpriv, [sk])
            plsc.store_scatter(mpriv, [sk], jnp.maximum(cur, sv), mask=last)
            return 0
        lax.fori_loop(0, _B // _L, group, 0)
        pltpu.sync_copy(lblk, logits_out.at[pl.ds(off, _B)])
        return 0
    lax.fori_loop(0, _NBLK, block, 0)

    pltpu.sync_copy(mpriv, shared_m.at[sid])
    plsc.subcore_barrier()
    soff = sid * _SLICE
    pltpu.sync_copy(shared_m.at[0, pl.ds(soff, _SLICE)], redacc)
    for t in range(1, _NS):
        pltpu.sync_copy(shared_m.at[t, pl.ds(soff, _SLICE)], redbuf)

        def redbody(i, _):
            redacc[pl.ds(i * _L, _L)] = jnp.maximum(
                redacc[pl.ds(i * _L, _L)], redbuf[pl.ds(i * _L, _L)])
            return 0
        lax.fori_loop(0, _SLICE // _L, redbody, 0)
    pltpu.sync_copy(redacc, msc_out.at[cid, pl.ds(soff, _SLICE)])


@functools.partial(
    pl.kernel,
    out_type=[jax.ShapeDtypeStruct((_NC, _NPAD, _D), jnp.float32),
              jax.ShapeDtypeStruct((_NC, _NS, _NPAD), jnp.float32),
              jax.ShapeDtypeStruct((_NC, _NPAD), jnp.float32)],
    mesh=_mesh,
    compiler_params=pltpu.CompilerParams(needs_layout_passes=False),
    scratch_types=[
        pltpu.VMEM((_B,), jnp.int32),
        pltpu.VMEM((_B,), jnp.int32),
        pltpu.VMEM((_B, _D), jnp.float32),
        pltpu.VMEM((_B,), jnp.float32),
        pltpu.VMEM((_NPAD,), jnp.float32),
        pltpu.VMEM((_NPAD,), jnp.float32),
        pltpu.VMEM((_SLICE,), jnp.float32),
        pltpu.VMEM((_SLICE,), jnp.float32),
        pltpu.VMEM((128, _D), jnp.float32),
        pltpu.VMEM_SHARED((_NPAD, _D), jnp.float32),
        pltpu.SemaphoreType.DMA,
    ],
)
def _sc_agg(v_hbm, src_hbm, dst_hbm, logits_hbm, m2_hbm, aggp_out, dstage_out,
            dsc_out, srcidx, dstidx, vrows, lblk, mloc, dloc, redacc, redbuf,
            zbuf, shared_agg, semv):
    cid = lax.axis_index("c")
    sid = lax.axis_index("s")
    wid = sid * _NC + cid
    base = wid * _EW
    iota = lax.iota(jnp.int32, _L)
    zero = jnp.zeros((_L,), jnp.float32)

    pltpu.sync_copy(m2_hbm.at[0], mloc)
    pltpu.sync_copy(m2_hbm.at[1], dloc)

    def maxbody(i, _):
        mloc[pl.ds(i * _L, _L)] = jnp.maximum(mloc[pl.ds(i * _L, _L)],
                                              dloc[pl.ds(i * _L, _L)])
        return 0
    lax.fori_loop(0, _NPAD // _L, maxbody, 0)

    def dzero(i, _):
        dloc[pl.ds(i * _L, _L)] = zero
        return 0
    lax.fori_loop(0, _NPAD // _L, dzero, 0)

    def zrow(r, _):
        for c in range(_D // _L):
            zbuf[r, pl.ds(c * _L, _L)] = zero
        return 0
    lax.fori_loop(0, 128, zrow, 0)
    for j in range(_SLICE // 128):
        pltpu.sync_copy(zbuf,
                        shared_agg.at[pl.ds(sid * _SLICE + j * 128, 128), :])
    plsc.subcore_barrier()

    def block(blk, _):
        off = base + blk * _B
        pltpu.sync_copy(src_hbm.at[pl.ds(off, _B)], srcidx)
        pltpu.sync_copy(dst_hbm.at[pl.ds(off, _B)], dstidx)
        pltpu.sync_copy(logits_hbm.at[pl.ds(off, _B)], lblk)
        cv = pltpu.async_copy(v_hbm.at[srcidx], vrows, semv)
        cv.wait()

        def group(g, _):
            gb = g * _L
            dstv = dstidx[pl.ds(gb, _L)]
            lv = lblk[pl.ds(gb, _L)]
            mg = plsc.load_gather(mloc, [dstv])
            ex = jnp.exp(lv - mg)
            for r in range(_L):
                e = gb + r
                exr = _take(ex, jnp.full((_L,), r, jnp.int32))
                for c in range(_D // _L):
                    vrows[e, pl.ds(c * _L, _L)] = (
                        vrows[e, pl.ds(c * _L, _L)] * exr)
            sk, sv = plsc.sort_key_val(dstv, ex)
            for s in (1, 2, 4, 8):
                kprev = _take(sk, jnp.maximum(iota - s, 0))
                vprev = _take(sv, jnp.maximum(iota - s, 0))
                same = (kprev == sk) & (iota >= s)
                sv = jnp.where(same, sv + vprev, sv)
            nxt = _take(sk, jnp.minimum(iota + 1, _L - 1))
            last = (sk != nxt) | (iota == _L - 1)
            plsc.addupdate_scatter(dloc, [sk], sv, mask=last)
            return 0
        lax.fori_loop(0, _B // _L, group, 0)
        pltpu.sync_copy(vrows, shared_agg.at[dstidx], add=True)
        return 0
    lax.fori_loop(0, _NBLK, block, 0)
    plsc.subcore_barrier()

    pltpu.sync_copy(shared_agg.at[pl.ds(sid * _SLICE, _SLICE), :],
                    aggp_out.at[cid, pl.ds(sid * _SLICE, _SLICE), :])

    pltpu.sync_copy(dloc, dstage_out.at[cid, sid])
    plsc.subcore_barrier()
    soff = sid * _SLICE
    pltpu.sync_copy(dstage_out.at[cid, 0, pl.ds(soff, _SLICE)], redacc)
    for t in range(1, _NS):
        pltpu.sync_copy(dstage_out.at[cid, t, pl.ds(soff, _SLICE)], redbuf)

        def redbody(i, _):
            redacc[pl.ds(i * _L, _L)] = (redacc[pl.ds(i * _L, _L)] +
                                         redbuf[pl.ds(i * _L, _L)])
            return 0
        lax.fori_loop(0, _SLICE // _L, redbody, 0)
    pltpu.sync_copy(redacc, dsc_out.at[cid, pl.ds(soff, _SLICE)])


def _proj_body(x_ref, w_ref, b_ref, o_ref):
    o_ref[...] = (jnp.dot(x_ref[...], w_ref[...],
                          preferred_element_type=jnp.float32) + b_ref[...])


def _proj(x, W, b):
    blk = 1000
    return pl.pallas_call(
        _proj_body,
        grid=(_N // blk,),
        in_specs=[pl.BlockSpec((blk, W.shape[0]), lambda i: (i, 0)),
                  pl.BlockSpec(W.shape, lambda i: (0, 0)),
                  pl.BlockSpec((1, W.shape[1]), lambda i: (0, 0))],
        out_specs=pl.BlockSpec((blk, W.shape[1]), lambda i: (i, 0)),
        out_shape=jax.ShapeDtypeStruct((_N, W.shape[1]), jnp.float32),
    )(x, W, b.reshape(1, -1))


def _comb_body(p0_ref, p1_ref, d0_ref, d1_ref, s_ref, w_ref, b_ref, o_ref):
    h = ((p0_ref[...] + p1_ref[...]) /
         (d0_ref[...] + d1_ref[...] + 1e-16) + s_ref[...])
    h = jnp.maximum(h, 0.0)
    o_ref[...] = (jnp.dot(h, w_ref[...],
                          preferred_element_type=jnp.float32) + b_ref[...])


def _comb_proj(p0, p1, d0, d1, skip, W, b):
    blk = 1000
    return pl.pallas_call(
        _comb_body,
        grid=(_N // blk,),
        in_specs=[pl.BlockSpec((blk, _D), lambda i: (i, 0)),
                  pl.BlockSpec((blk, _D), lambda i: (i, 0)),
                  pl.BlockSpec((blk, 1), lambda i: (i, 0)),
                  pl.BlockSpec((blk, 1), lambda i: (i, 0)),
                  pl.BlockSpec((blk, _D), lambda i: (i, 0)),
                  pl.BlockSpec(W.shape, lambda i: (0, 0)),
                  pl.BlockSpec((1, W.shape[1]), lambda i: (0, 0))],
        out_specs=pl.BlockSpec((blk, W.shape[1]), lambda i: (i, 0)),
        out_shape=jax.ShapeDtypeStruct((_N, W.shape[1]), jnp.float32),
    )(p0, p1, d0, d1, skip, W, b.reshape(1, -1))


def _pool_body(p0_ref, p1_ref, d0_ref, d1_ref, s_ref, batch_ref, o_ref,
               cnt_ref):
    i = pl.program_id(0)

    @pl.when(i == 0)
    def _():
        o_ref[...] = jnp.zeros_like(o_ref)
        cnt_ref[...] = jnp.zeros_like(cnt_ref)

    h = ((p0_ref[...] + p1_ref[...]) /
         (d0_ref[...] + d1_ref[...] + 1e-16) + s_ref[...])
    row = batch_ref[...].reshape(1, -1)
    gid = lax.broadcasted_iota(jnp.int32, (_G, row.shape[1]), 0)
    oh = (gid == row).astype(jnp.float32)
    o_ref[...] += jnp.dot(oh, h, preferred_element_type=jnp.float32)
    cnt_ref[...] += jnp.sum(oh, axis=1, keepdims=True)

    @pl.when(i == pl.num_programs(0) - 1)
    def _():
        o_ref[...] = o_ref[...] / jnp.maximum(cnt_ref[...], 1.0)


def _pool(p0, p1, d0, d1, skip, batch2d):
    blk = 1000
    return pl.pallas_call(
        _pool_body,
        grid=(_N // blk,),
        in_specs=[pl.BlockSpec((blk, _D), lambda i: (i, 0)),
                  pl.BlockSpec((blk, _D), lambda i: (i, 0)),
                  pl.BlockSpec((blk, 1), lambda i: (i, 0)),
                  pl.BlockSpec((blk, 1), lambda i: (i, 0)),
                  pl.BlockSpec((blk, _D), lambda i: (i, 0)),
                  pl.BlockSpec((1, 1, blk), lambda i: (i, 0, 0))],
        out_specs=pl.BlockSpec((_G, _D), lambda i: (0, 0)),
        out_shape=jax.ShapeDtypeStruct((_G, _D), jnp.float32),
        scratch_shapes=[pltpu.VMEM((_G, _D), jnp.float32)],
    )(p0, p1, d0, d1, skip, batch2d)


def kernel(x, edge_index, edge_attr, batch, Wq1, bq1, Wk1, bk1, Wv1, bv1,
           Ws1, bs1, Wq2, bq2, Wk2, bk2, Wv2, bv2, Ws2, bs2):
    x = x.astype(jnp.float32)
    src = edge_index[0]
    dst = edge_index[1]

    W1 = jnp.concatenate([Wq1, Wk1, Wv1, Ws1], axis=1)
    b1 = jnp.concatenate([bq1, bk1, bv1, bs1])
    p1 = _proj(x, W1, b1)
    q1, k1, v1, s1 = (p1[:, :_D], p1[:, _D:2 * _D],
                      p1[:, 2 * _D:3 * _D], p1[:, 3 * _D:])

    logits1, m1 = _sc_logits_max(q1, k1, src, dst)
    aggp1, _dstage1, d1 = _sc_agg(v1, src, dst, logits1, m1)

    W2 = jnp.concatenate([Wq2, Wk2, Wv2, Ws2], axis=1)
    b2 = jnp.concatenate([bq2, bk2, bv2, bs2])
    p2 = _comb_proj(aggp1[0, :_N], aggp1[1, :_N],
                    d1[0, :_N, None], d1[1, :_N, None], s1, W2, b2)
    q2, k2, v2, s2 = (p2[:, :_D], p2[:, _D:2 * _D],
                      p2[:, 2 * _D:3 * _D], p2[:, 3 * _D:])

    logits2, m2 = _sc_logits_max(q2, k2, src, dst)
    aggp2, _dstage2, d2 = _sc_agg(v2, src, dst, logits2, m2)

    return _pool(aggp2[0, :_N], aggp2[1, :_N],
                 d2[0, :_N, None], d2[1, :_N, None], s2,
                 batch.reshape(10, 1, _N // 10))


# speedup vs baseline: 14.3763x; 1.8871x over previous
import functools

import jax
import jax.numpy as jnp
from jax import lax
from jax.experimental import pallas as pl
from jax.experimental.pallas import tpu as pltpu
from jax.experimental.pallas import tpu_sc as plsc

_N = 10000
_E = 320000
_D = 128
_G = 256

_NC = 2
_NS = 16
_NW = _NC * _NS
_L = 16

_NPAD = 10240
_SLICE = _NPAD // _NS
_EW = _E // _NW
_B = 80
_NBLK = _EW // _B
_RS = _N // _NS

_NEG = -3.0e38
_SCALE = 1.0 / (128.0 ** 0.5)

_mesh = plsc.VectorSubcoreMesh(core_axis_name="c", subcore_axis_name="s",
                               num_cores=_NC, num_subcores=_NS)


def _take(x, idx):
    return jnp.take_along_axis(x, idx, axis=0)


@functools.partial(
    pl.kernel,
    out_type=[jax.ShapeDtypeStruct((_NW, _NBLK, _B), jnp.float32),
              jax.ShapeDtypeStruct((_NC, _NPAD), jnp.float32)],
    mesh=_mesh,
    compiler_params=pltpu.CompilerParams(needs_layout_passes=False),
    scratch_types=[
        pltpu.VMEM((_NBLK, _B), jnp.int32),
        pltpu.VMEM((_NBLK, _B), jnp.int32),
        pltpu.VMEM((_B, _D), jnp.float32),
        pltpu.VMEM((_B, _D), jnp.float32),
        pltpu.VMEM((_B, _D), jnp.float32),
        pltpu.VMEM((_B, _D), jnp.float32),
        pltpu.VMEM((_NBLK, _B), jnp.float32),
        pltpu.VMEM((_L * _L,), jnp.float32),
        pltpu.VMEM((_NPAD,), jnp.float32),
        pltpu.VMEM((_SLICE,), jnp.float32),
        pltpu.VMEM((_SLICE,), jnp.float32),
        pltpu.VMEM_SHARED((_NS, _NPAD), jnp.float32),
        pltpu.SemaphoreType.DMA,
        pltpu.SemaphoreType.DMA,
        pltpu.SemaphoreType.DMA,
        pltpu.SemaphoreType.DMA,
    ],
)
def _sc_logits_max(q_hbm, k_hbm, src_hbm, dst_hbm, logits_out, msc_out,
                   srcall, dstall, qrows0, krows0, qrows1, krows1, lall,
                   accm, mpriv, redacc, redbuf, shared_m,
                   semq0, semk0, semq1, semk1):
    cid = lax.axis_index("c")
    sid = lax.axis_index("s")
    wid = sid * _NC + cid
    iota = lax.iota(jnp.int32, _L)
    neg = jnp.full((_L,), _NEG, jnp.float32)

    def initbody(i, _):
        mpriv[pl.ds(i * _L, _L)] = neg
        return 0
    lax.fori_loop(0, _NPAD // _L, initbody, 0)

    pltpu.sync_copy(src_hbm.at[wid], srcall)
    pltpu.sync_copy(dst_hbm.at[wid], dstall)

    def issue(blk, qb, kb, sq, sk_):
        pltpu.async_copy(q_hbm.at[dstall.at[blk]], qb, sq)
        pltpu.async_copy(k_hbm.at[srcall.at[blk]], kb, sk_)

    def wait_rows(buf, sem):
        pltpu.make_async_copy(q_hbm.at[pl.ds(0, _B), :], buf, sem).wait()

    def compute(blk, qb, kb):
        def group(g, _):
            gb = g * _L
            for r in range(_L):
                e = gb + r
                acc = qb[e, pl.ds(0, _L)] * kb[e, pl.ds(0, _L)]
                for c in range(1, _D // _L):
                    acc = acc + (qb[e, pl.ds(c * _L, _L)] *
                                 kb[e, pl.ds(c * _L, _L)])
                accm[pl.ds(r * _L, _L)] = acc
            tot = plsc.load_gather(accm, [iota * _L])
            for l in range(1, _L):
                tot = tot + plsc.load_gather(accm, [iota * _L + l])
            lv = tot * _SCALE
            lall[blk, pl.ds(gb, _L)] = lv

            dstv = dstall[blk, pl.ds(gb, _L)]
            sk, sv = plsc.sort_key_val(dstv, lv)
            for s in (1, 2, 4, 8):
                kprev = _take(sk, jnp.maximum(iota - s, 0))
                vprev = _take(sv, jnp.maximum(iota - s, 0))
                same = (kprev == sk) & (iota >= s)
                sv = jnp.where(same, jnp.maximum(sv, vprev), sv)
            nxt = _take(sk, jnp.minimum(iota + 1, _L - 1))
            last = (sk != nxt) | (iota == _L - 1)
            cur = plsc.load_gather(mpriv, [sk])
            plsc.store_scatter(mpriv, [sk], jnp.maximum(cur, sv), mask=last)
            return 0
        lax.fori_loop(0, _B // _L, group, 0)

    issue(0, qrows0, krows0, semq0, semk0)
    issue(1, qrows1, krows1, semq1, semk1)
    wait_rows(qrows0, semq0)
    wait_rows(krows0, semk0)
    compute(0, qrows0, krows0)

    def pair(k, _):
        a = 2 * k + 1
        issue(a + 1, qrows0, krows0, semq0, semk0)
        wait_rows(qrows1, semq1)
        wait_rows(krows1, semk1)
        compute(a, qrows1, krows1)

        @pl.when(k != (_NBLK - 3) // 2)
        def _():
            issue(a + 2, qrows1, krows1, semq1, semk1)
        wait_rows(qrows0, semq0)
        wait_rows(krows0, semk0)
        compute(a + 1, qrows0, krows0)
        return 0
    lax.fori_loop(0, (_NBLK - 1) // 2, pair, 0)
    pltpu.sync_copy(lall, logits_out.at[wid])

    pltpu.sync_copy(mpriv, shared_m.at[sid])
    plsc.subcore_barrier()
    soff = sid * _SLICE
    pltpu.sync_copy(shared_m.at[0, pl.ds(soff, _SLICE)], redacc)
    for t in range(1, _NS):
        pltpu.sync_copy(shared_m.at[t, pl.ds(soff, _SLICE)], redbuf)

        def redbody(i, _):
            redacc[pl.ds(i * _L, _L)] = jnp.maximum(
                redacc[pl.ds(i * _L, _L)], redbuf[pl.ds(i * _L, _L)])
            return 0
        lax.fori_loop(0, _SLICE // _L, redbody, 0)
    pltpu.sync_copy(redacc, msc_out.at[cid, pl.ds(soff, _SLICE)])


@functools.partial(
    pl.kernel,
    out_type=jax.ShapeDtypeStruct((_NW, _NBLK, _B), jnp.float32),
    mesh=_mesh,
    compiler_params=pltpu.CompilerParams(needs_layout_passes=False),
    scratch_types=[
        pltpu.VMEM((_NBLK, _B), jnp.int32),
        pltpu.VMEM((_NBLK, _B), jnp.float32),
        pltpu.VMEM((_NPAD,), jnp.float32),
        pltpu.VMEM((_NPAD,), jnp.float32),
    ],
)
def _sc_subm(logits_hbm, dst_hbm, m2_hbm, ldiff_out, dstall, lall, mloc, mtmp):
    cid = lax.axis_index("c")
    sid = lax.axis_index("s")
    wid = sid * _NC + cid

    pltpu.sync_copy(m2_hbm.at[0], mloc)
    pltpu.sync_copy(m2_hbm.at[1], mtmp)

    def maxbody(i, _):
        mloc[pl.ds(i * _L, _L)] = jnp.maximum(mloc[pl.ds(i * _L, _L)],
                                              mtmp[pl.ds(i * _L, _L)])
        return 0
    lax.fori_loop(0, _NPAD // _L, maxbody, 0)

    pltpu.sync_copy(logits_hbm.at[wid], lall)
    pltpu.sync_copy(dst_hbm.at[wid], dstall)

    def block(blk, _):
        def group(g, _):
            gb = g * _L
            dstv = dstall[blk, pl.ds(gb, _L)]
            mg = plsc.load_gather(mloc, [dstv])
            lall[blk, pl.ds(gb, _L)] = lall[blk, pl.ds(gb, _L)] - mg
            return 0
        lax.fori_loop(0, _B // _L, group, 0)
        return 0
    lax.fori_loop(0, _NBLK, block, 0)
    pltpu.sync_copy(lall, ldiff_out.at[wid])


_CH = 25
_NCHUNK = _NBLK // _CH


@functools.partial(
    pl.kernel,
    out_type=[jax.ShapeDtypeStruct((_NC, _NPAD, _D), jnp.float32),
              jax.ShapeDtypeStruct((_NC, _NS, _NPAD), jnp.float32),
              jax.ShapeDtypeStruct((_NC, _NPAD), jnp.float32)],
    mesh=_mesh,
    compiler_params=pltpu.CompilerParams(needs_layout_passes=False),
    scratch_types=[
        pltpu.VMEM((_CH, _B), jnp.int32),
        pltpu.VMEM((_CH, _B), jnp.int32),
        pltpu.VMEM((_CH, _B), jnp.float32),
        pltpu.VMEM((_B, _D), jnp.float32),
        pltpu.VMEM((_B, _D), jnp.float32),
        pltpu.VMEM((_NPAD,), jnp.float32),
        pltpu.VMEM((_SLICE,), jnp.float32),
        pltpu.VMEM((_SLICE,), jnp.float32),
        pltpu.VMEM_SHARED((_NPAD, _D), jnp.float32),
        pltpu.SemaphoreType.DMA,
        pltpu.SemaphoreType.DMA,
        pltpu.SemaphoreType.DMA,
        pltpu.SemaphoreType.DMA,
    ],
)
def _sc_agg(v_hbm, src_hbm, dst_hbm, logits_hbm, aggp_out, dstage_out,
            dsc_out, srcch, dstch, lch, vrows0, vrows1, dloc,
            redacc, redbuf, shared_agg, semg0, semg1, sems0, sems1):
    cid = lax.axis_index("c")
    sid = lax.axis_index("s")
    wid = sid * _NC + cid
    iota = lax.iota(jnp.int32, _L)
    zero = jnp.zeros((_L,), jnp.float32)

    def dzero(i, _):
        dloc[pl.ds(i * _L, _L)] = zero
        return 0
    lax.fori_loop(0, _NPAD // _L, dzero, 0)

    def vzero(r, _):
        for c in range(_D // _L):
            vrows0[r, pl.ds(c * _L, _L)] = zero
            vrows1[r, pl.ds(c * _L, _L)] = zero
        return 0
    lax.fori_loop(0, _B, vzero, 0)
    for j in range(_SLICE // _B):
        pltpu.sync_copy(vrows0,
                        shared_agg.at[pl.ds(sid * _SLICE + j * _B, _B), :])
    plsc.subcore_barrier()

    def issue_g(blk, vb, sem):
        pltpu.async_copy(v_hbm.at[srcch.at[blk]], vb, sem)

    def wait_rows(buf, sem):
        pltpu.make_async_copy(v_hbm.at[pl.ds(0, _B), :], buf, sem).wait()

    def issue_s(blk, vb, sem):
        pltpu.async_copy(vb, shared_agg.at[dstch.at[blk]], sem, add=True)

    def wait_s(vb, sem):
        pltpu.make_async_copy(vb, shared_agg.at[pl.ds(0, _B), :], sem).wait()

    def compute(blk, vb):
        def group(g, _):
            gb = g * _L
            dstv = dstch[blk, pl.ds(gb, _L)]
            lv = lch[blk, pl.ds(gb, _L)]
            ex = jnp.exp(lv)
            for r in range(_L):
                e = gb + r
                exr = _take(ex, jnp.full((_L,), r, jnp.int32))
                for c in range(_D // _L):
                    vb[e, pl.ds(c * _L, _L)] = vb[e, pl.ds(c * _L, _L)] * exr
            sk, sv = plsc.sort_key_val(dstv, ex)
            for s in (1, 2, 4, 8):
                kprev = _take(sk, jnp.maximum(iota - s, 0))
                vprev = _take(sv, jnp.maximum(iota - s, 0))
                same = (kprev == sk) & (iota >= s)
                sv = jnp.where(same, sv + vprev, sv)
            nxt = _take(sk, jnp.minimum(iota + 1, _L - 1))
            last = (sk != nxt) | (iota == _L - 1)
            plsc.addupdate_scatter(dloc, [sk], sv, mask=last)
            return 0
        lax.fori_loop(0, _B // _L, group, 0)

    def chunk(c, _):
        pltpu.sync_copy(src_hbm.at[wid, c], srcch)
        pltpu.sync_copy(dst_hbm.at[wid, c], dstch)
        pltpu.sync_copy(logits_hbm.at[wid, c], lch)

        issue_g(0, vrows0, semg0)
        issue_g(1, vrows1, semg1)
        wait_rows(vrows0, semg0)
        compute(0, vrows0)
        issue_s(0, vrows0, sems0)

        def pair(k, _):
            a = 2 * k + 1
            wait_rows(vrows1, semg1)
            compute(a, vrows1)
            wait_s(vrows0, sems0)
            issue_g(a + 1, vrows0, semg0)
            issue_s(a, vrows1, sems1)

            wait_rows(vrows0, semg0)
            compute(a + 1, vrows0)
            wait_s(vrows1, sems1)

            @pl.when(k != (_CH - 3) // 2)
            def _():
                issue_g(a + 2, vrows1, semg1)
            issue_s(a + 1, vrows0, sems0)
            return 0
        lax.fori_loop(0, (_CH - 1) // 2, pair, 0)
        wait_s(vrows0, sems0)
        return 0
    lax.fori_loop(0, _NCHUNK, chunk, 0)
    plsc.subcore_barrier()

    pltpu.sync_copy(shared_agg.at[pl.ds(sid * _SLICE, _SLICE), :],
                    aggp_out.at[cid, pl.ds(sid * _SLICE, _SLICE), :])

    pltpu.sync_copy(dloc, dstage_out.at[cid, sid])
    plsc.subcore_barrier()
    soff = sid * _SLICE
    pltpu.sync_copy(dstage_out.at[cid, 0, pl.ds(soff, _SLICE)], redacc)
    for t in range(1, _NS):
        pltpu.sync_copy(dstage_out.at[cid, t, pl.ds(soff, _SLICE)], redbuf)

        def redbody(i, _):
            redacc[pl.ds(i * _L, _L)] = (redacc[pl.ds(i * _L, _L)] +
                                         redbuf[pl.ds(i * _L, _L)])
            return 0
        lax.fori_loop(0, _SLICE // _L, redbody, 0)
    pltpu.sync_copy(redacc, dsc_out.at[cid, pl.ds(soff, _SLICE)])


def _proj_body(x_ref, w_ref, b_ref, o_ref):
    o_ref[...] = (jnp.dot(x_ref[...], w_ref[...],
                          preferred_element_type=jnp.float32) + b_ref[...])


def _proj(x, W, b):
    blk = 1000
    return pl.pallas_call(
        _proj_body,
        grid=(_N // blk,),
        in_specs=[pl.BlockSpec((blk, W.shape[0]), lambda i: (i, 0)),
                  pl.BlockSpec(W.shape, lambda i: (0, 0)),
                  pl.BlockSpec((1, W.shape[1]), lambda i: (0, 0))],
        out_specs=pl.BlockSpec((blk, W.shape[1]), lambda i: (i, 0)),
        out_shape=jax.ShapeDtypeStruct((_N, W.shape[1]), jnp.float32),
    )(x, W, b.reshape(1, -1))


def _comb_body(p0_ref, p1_ref, d0_ref, d1_ref, s_ref, w_ref, b_ref, o_ref):
    h = ((p0_ref[...] + p1_ref[...]) /
         (d0_ref[...] + d1_ref[...] + 1e-16) + s_ref[...])
    h = jnp.maximum(h, 0.0)
    o_ref[...] = (jnp.dot(h, w_ref[...],
                          preferred_element_type=jnp.float32) + b_ref[...])


def _comb_proj(p0, p1, d0, d1, skip, W, b):
    blk = 1000
    return pl.pallas_call(
        _comb_body,
        grid=(_N // blk,),
        in_specs=[pl.BlockSpec((blk, _D), lambda i: (i, 0)),
                  pl.BlockSpec((blk, _D), lambda i: (i, 0)),
                  pl.BlockSpec((blk, 1), lambda i: (i, 0)),
                  pl.BlockSpec((blk, 1), lambda i: (i, 0)),
                  pl.BlockSpec((blk, _D), lambda i: (i, 0)),
                  pl.BlockSpec(W.shape, lambda i: (0, 0)),
                  pl.BlockSpec((1, W.shape[1]), lambda i: (0, 0))],
        out_specs=pl.BlockSpec((blk, W.shape[1]), lambda i: (i, 0)),
        out_shape=jax.ShapeDtypeStruct((_N, W.shape[1]), jnp.float32),
    )(p0, p1, d0, d1, skip, W, b.reshape(1, -1))


def _pool_body(p0_ref, p1_ref, d0_ref, d1_ref, s_ref, batch_ref, o_ref,
               cnt_ref):
    i = pl.program_id(0)

    @pl.when(i == 0)
    def _():
        o_ref[...] = jnp.zeros_like(o_ref)
        cnt_ref[...] = jnp.zeros_like(cnt_ref)

    h = ((p0_ref[...] + p1_ref[...]) /
         (d0_ref[...] + d1_ref[...] + 1e-16) + s_ref[...])
    row = batch_ref[...].reshape(1, -1)
    gid = lax.broadcasted_iota(jnp.int32, (_G, row.shape[1]), 0)
    oh = (gid == row).astype(jnp.float32)
    o_ref[...] += jnp.dot(oh, h, preferred_element_type=jnp.float32)
    cnt_ref[...] += jnp.sum(oh, axis=1, keepdims=True)

    @pl.when(i == pl.num_programs(0) - 1)
    def _():
        o_ref[...] = o_ref[...] / jnp.maximum(cnt_ref[...], 1.0)


def _pool(p0, p1, d0, d1, skip, batch2d):
    blk = 1000
    return pl.pallas_call(
        _pool_body,
        grid=(_N // blk,),
        in_specs=[pl.BlockSpec((blk, _D), lambda i: (i, 0)),
                  pl.BlockSpec((blk, _D), lambda i: (i, 0)),
                  pl.BlockSpec((blk, 1), lambda i: (i, 0)),
                  pl.BlockSpec((blk, 1), lambda i: (i, 0)),
                  pl.BlockSpec((blk, _D), lambda i: (i, 0)),
                  pl.BlockSpec((1, 1, blk), lambda i: (i, 0, 0))],
        out_specs=pl.BlockSpec((_G, _D), lambda i: (0, 0)),
        out_shape=jax.ShapeDtypeStruct((_G, _D), jnp.float32),
        scratch_shapes=[pltpu.VMEM((_G, _D), jnp.float32)],
    )(p0, p1, d0, d1, skip, batch2d)


def kernel(x, edge_index, edge_attr, batch, Wq1, bq1, Wk1, bk1, Wv1, bv1,
           Ws1, bs1, Wq2, bq2, Wk2, bk2, Wv2, bv2, Ws2, bs2):
    x = x.astype(jnp.float32)
    src = edge_index[0].reshape(_NW, _NBLK, _B)
    dst = edge_index[1].reshape(_NW, _NBLK, _B)

    W1 = jnp.concatenate([Wq1, Wk1, Wv1, Ws1], axis=1)
    b1 = jnp.concatenate([bq1, bk1, bv1, bs1])
    p1 = _proj(x, W1, b1)
    q1, k1, v1, s1 = (p1[:, :_D], p1[:, _D:2 * _D],
                      p1[:, 2 * _D:3 * _D], p1[:, 3 * _D:])

    logits1, m1 = _sc_logits_max(q1, k1, src, dst)
    src4 = src.reshape(_NW, _NCHUNK, _CH, _B)
    dst4 = dst.reshape(_NW, _NCHUNK, _CH, _B)
    ldiff1 = _sc_subm(logits1, dst, m1)
    aggp1, _dstage1, d1 = _sc_agg(
        v1, src4, dst4, ldiff1.reshape(_NW, _NCHUNK, _CH, _B))

    W2 = jnp.concatenate([Wq2, Wk2, Wv2, Ws2], axis=1)
    b2 = jnp.concatenate([bq2, bk2, bv2, bs2])
    p2 = _comb_proj(aggp1[0, :_N], aggp1[1, :_N],
                    d1[0, :_N, None], d1[1, :_N, None], s1, W2, b2)
    q2, k2, v2, s2 = (p2[:, :_D], p2[:, _D:2 * _D],
                      p2[:, 2 * _D:3 * _D], p2[:, 3 * _D:])

    logits2, m2 = _sc_logits_max(q2, k2, src, dst)
    ldiff2 = _sc_subm(logits2, dst, m2)
    aggp2, _dstage2, d2 = _sc_agg(
        v2, src4, dst4, ldiff2.reshape(_NW, _NCHUNK, _CH, _B))

    return _pool(aggp2[0, :_N], aggp2[1, :_N],
                 d2[0, :_N, None], d2[1, :_N, None], s2,
                 batch.reshape(10, 1, _N // 10))


# speedup vs baseline: 14.8198x; 1.0308x over previous
import functools

import jax
import jax.numpy as jnp
from jax import lax
from jax.experimental import pallas as pl
from jax.experimental.pallas import tpu as pltpu
from jax.experimental.pallas import tpu_sc as plsc

_N = 10000
_E = 320000
_D = 128
_G = 256

_NC = 2
_NS = 16
_NW = _NC * _NS
_L = 16

_NPAD = 10240
_SLICE = _NPAD // _NS
_EW = _E // _NW
_B = 80
_NBLK = _EW // _B
_RS = _N // _NS

_NEG = -3.0e38
_SCALE = 1.0 / (128.0 ** 0.5)

_mesh = plsc.VectorSubcoreMesh(core_axis_name="c", subcore_axis_name="s",
                               num_cores=_NC, num_subcores=_NS)


def _take(x, idx):
    return jnp.take_along_axis(x, idx, axis=0)


@functools.partial(
    pl.kernel,
    out_type=[jax.ShapeDtypeStruct((_NW, _NBLK, _B), jnp.float32),
              jax.ShapeDtypeStruct((_NC, _NPAD), jnp.float32)],
    mesh=_mesh,
    compiler_params=pltpu.CompilerParams(needs_layout_passes=False),
    scratch_types=[
        pltpu.VMEM((_NBLK, _B), jnp.int32),
        pltpu.VMEM((_NBLK, _B), jnp.int32),
        pltpu.VMEM((_B, _D), jnp.float32),
        pltpu.VMEM((_B, _D), jnp.float32),
        pltpu.VMEM((_B, _D), jnp.float32),
        pltpu.VMEM((_B, _D), jnp.float32),
        pltpu.VMEM((_NBLK, _B), jnp.float32),
        pltpu.VMEM((_L * _L,), jnp.float32),
        pltpu.VMEM((_NPAD,), jnp.float32),
        pltpu.VMEM((_SLICE,), jnp.float32),
        pltpu.VMEM((_NS, _SLICE), jnp.float32),
        pltpu.VMEM_SHARED((_NS, _NPAD), jnp.float32),
        pltpu.SemaphoreType.DMA,
        pltpu.SemaphoreType.DMA,
        pltpu.SemaphoreType.DMA,
        pltpu.SemaphoreType.DMA,
    ],
)
def _sc_logits_max(q_hbm, k_hbm, src_hbm, dst_hbm, logits_out, msc_out,
                   srcall, dstall, qrows0, krows0, qrows1, krows1, lall,
                   accm, mpriv, redacc, redbufs, shared_m,
                   semq0, semk0, semq1, semk1):
    cid = lax.axis_index("c")
    sid = lax.axis_index("s")
    wid = sid * _NC + cid
    iota = lax.iota(jnp.int32, _L)
    neg = jnp.full((_L,), _NEG, jnp.float32)

    def initbody(i, _):
        mpriv[pl.ds(i * _L, _L)] = neg
        return 0
    lax.fori_loop(0, _NPAD // _L, initbody, 0)

    pltpu.sync_copy(src_hbm.at[wid], srcall)
    pltpu.sync_copy(dst_hbm.at[wid], dstall)

    def issue(blk, qb, kb, sq, sk_):
        pltpu.async_copy(q_hbm.at[dstall.at[blk]], qb, sq)
        pltpu.async_copy(k_hbm.at[srcall.at[blk]], kb, sk_)

    def wait_rows(buf, sem):
        pltpu.make_async_copy(q_hbm.at[pl.ds(0, _B), :], buf, sem).wait()

    def wait_rows_k(buf, sem):
        pltpu.make_async_copy(k_hbm.at[pl.ds(0, _B), :], buf, sem).wait()

    def compute(blk, qb, kb):
        def group(g, _):
            gb = g * _L
            for r in range(_L):
                e = gb + r
                acc = qb[e, pl.ds(0, _L)] * kb[e, pl.ds(0, _L)]
                for c in range(1, _D // _L):
                    acc = acc + (qb[e, pl.ds(c * _L, _L)] *
                                 kb[e, pl.ds(c * _L, _L)])
                accm[pl.ds(r * _L, _L)] = acc
            tot = plsc.load_gather(accm, [iota * _L])
            for l in range(1, _L):
                tot = tot + plsc.load_gather(accm, [iota * _L + l])
            lv = tot * _SCALE
            lall[blk, pl.ds(gb, _L)] = lv

            dstv = dstall[blk, pl.ds(gb, _L)]
            sk, sv = plsc.sort_key_val(dstv, lv)
            for s in (1, 2, 4, 8):
                kprev = _take(sk, jnp.maximum(iota - s, 0))
                vprev = _take(sv, jnp.maximum(iota - s, 0))
                same = (kprev == sk) & (iota >= s)
                sv = jnp.where(same, jnp.maximum(sv, vprev), sv)
            nxt = _take(sk, jnp.minimum(iota + 1, _L - 1))
            last = (sk != nxt) | (iota == _L - 1)
            cur = plsc.load_gather(mpriv, [sk])
            plsc.store_scatter(mpriv, [sk], jnp.maximum(cur, sv), mask=last)
            return 0
        lax.fori_loop(0, _B // _L, group, 0)

    issue(0, qrows0, krows0, semq0, semk0)
    issue(1, qrows1, krows1, semq1, semk1)
    wait_rows(qrows0, semq0)
    wait_rows_k(krows0, semk0)
    compute(0, qrows0, krows0)

    def pair(k, _):
        a = 2 * k + 1
        issue(a + 1, qrows0, krows0, semq0, semk0)
        wait_rows(qrows1, semq1)
        wait_rows_k(krows1, semk1)
        compute(a, qrows1, krows1)

        @pl.when(k != (_NBLK - 3) // 2)
        def _():
            issue(a + 2, qrows1, krows1, semq1, semk1)
        wait_rows(qrows0, semq0)
        wait_rows_k(krows0, semk0)
        compute(a + 1, qrows0, krows0)
        return 0
    lax.fori_loop(0, (_NBLK - 1) // 2, pair, 0)
    pltpu.sync_copy(lall, logits_out.at[wid])

    pltpu.sync_copy(mpriv, shared_m.at[sid])
    plsc.subcore_barrier()
    soff = sid * _SLICE
    for t in range(_NS):
        pltpu.async_copy(shared_m.at[t, pl.ds(soff, _SLICE)],
                         redbufs.at[t], semq0)
    for t in range(_NS):
        pltpu.make_async_copy(shared_m.at[0, pl.ds(soff, _SLICE)],
                              redbufs.at[0], semq0).wait()

    def redbody(i, _):
        v = redbufs[0, pl.ds(i * _L, _L)]
        for t in range(1, _NS):
            v = jnp.maximum(v, redbufs[t, pl.ds(i * _L, _L)])
        redacc[pl.ds(i * _L, _L)] = v
        return 0
    lax.fori_loop(0, _SLICE // _L, redbody, 0)
    pltpu.sync_copy(redacc, msc_out.at[cid, pl.ds(soff, _SLICE)])


@functools.partial(
    pl.kernel,
    out_type=jax.ShapeDtypeStruct((_NW, _NBLK, _B), jnp.float32),
    mesh=_mesh,
    compiler_params=pltpu.CompilerParams(needs_layout_passes=False),
    scratch_types=[
        pltpu.VMEM((_NBLK, _B), jnp.int32),
        pltpu.VMEM((_NBLK, _B), jnp.float32),
        pltpu.VMEM((_NPAD,), jnp.float32),
        pltpu.VMEM((_NPAD,), jnp.float32),
    ],
)
def _sc_subm(logits_hbm, dst_hbm, m2_hbm, ldiff_out, dstall, lall, mloc, mtmp):
    cid = lax.axis_index("c")
    sid = lax.axis_index("s")
    wid = sid * _NC + cid

    pltpu.sync_copy(m2_hbm.at[0], mloc)
    pltpu.sync_copy(m2_hbm.at[1], mtmp)

    def maxbody(i, _):
        mloc[pl.ds(i * _L, _L)] = jnp.maximum(mloc[pl.ds(i * _L, _L)],
                                              mtmp[pl.ds(i * _L, _L)])
        return 0
    lax.fori_loop(0, _NPAD // _L, maxbody, 0)

    pltpu.sync_copy(logits_hbm.at[wid], lall)
    pltpu.sync_copy(dst_hbm.at[wid], dstall)

    def block(blk, _):
        def group(g, _):
            gb = g * _L
            dstv = dstall[blk, pl.ds(gb, _L)]
            mg = plsc.load_gather(mloc, [dstv])
            lall[blk, pl.ds(gb, _L)] = lall[blk, pl.ds(gb, _L)] - mg
            return 0
        lax.fori_loop(0, _B // _L, group, 0)
        return 0
    lax.fori_loop(0, _NBLK, block, 0)
    pltpu.sync_copy(lall, ldiff_out.at[wid])


_CH = 25
_NCHUNK = _NBLK // _CH


@functools.partial(
    pl.kernel,
    out_type=[jax.ShapeDtypeStruct((_NC, _NPAD, _D), jnp.float32),
              jax.ShapeDtypeStruct((_NC, _NS, _NPAD), jnp.float32),
              jax.ShapeDtypeStruct((_NC, _NPAD), jnp.float32)],
    mesh=_mesh,
    compiler_params=pltpu.CompilerParams(needs_layout_passes=False),
    scratch_types=[
        pltpu.VMEM((_CH, _B), jnp.int32),
        pltpu.VMEM((_CH, _B), jnp.int32),
        pltpu.VMEM((_CH, _B), jnp.float32),
        pltpu.VMEM((_B, _D), jnp.float32),
        pltpu.VMEM((_B, _D), jnp.float32),
        pltpu.VMEM((_NPAD,), jnp.float32),
        pltpu.VMEM((_SLICE,), jnp.float32),
        pltpu.VMEM((4, _SLICE), jnp.float32),
        pltpu.VMEM_SHARED((_NPAD, _D), jnp.float32),
        pltpu.SemaphoreType.DMA,
        pltpu.SemaphoreType.DMA,
        pltpu.SemaphoreType.DMA,
        pltpu.SemaphoreType.DMA,
    ],
)
def _sc_agg(v_hbm, src_hbm, dst_hbm, logits_hbm, aggp_out, dstage_out,
            dsc_out, srcch, dstch, lch, vrows0, vrows1, dloc,
            redacc, redbufs4, shared_agg, semg0, semg1, sems0, sems1):
    cid = lax.axis_index("c")
    sid = lax.axis_index("s")
    wid = sid * _NC + cid
    iota = lax.iota(jnp.int32, _L)
    zero = jnp.zeros((_L,), jnp.float32)

    def dzero(i, _):
        dloc[pl.ds(i * _L, _L)] = zero
        return 0
    lax.fori_loop(0, _NPAD // _L, dzero, 0)

    def vzero(r, _):
        for c in range(_D // _L):
            vrows0[r, pl.ds(c * _L, _L)] = zero
            vrows1[r, pl.ds(c * _L, _L)] = zero
        return 0
    lax.fori_loop(0, _B, vzero, 0)
    for j in range(_SLICE // _B):
        pltpu.sync_copy(vrows0,
                        shared_agg.at[pl.ds(sid * _SLICE + j * _B, _B), :])
    plsc.subcore_barrier()

    def issue_g(blk, vb, sem):
        pltpu.async_copy(v_hbm.at[srcch.at[blk]], vb, sem)

    def wait_rows(buf, sem):
        pltpu.make_async_copy(v_hbm.at[pl.ds(0, _B), :], buf, sem).wait()

    def issue_s(blk, vb, sem):
        pltpu.async_copy(vb, shared_agg.at[dstch.at[blk]], sem, add=True)

    def wait_s(vb, sem):
        pltpu.make_async_copy(vb, shared_agg.at[pl.ds(0, _B), :], sem).wait()

    def compute(blk, vb):
        def group(g, _):
            gb = g * _L
            dstv = dstch[blk, pl.ds(gb, _L)]
            lv = lch[blk, pl.ds(gb, _L)]
            ex = jnp.exp(lv)
            for r in range(_L):
                e = gb + r
                exr = _take(ex, jnp.full((_L,), r, jnp.int32))
                for c in range(_D // _L):
                    vb[e, pl.ds(c * _L, _L)] = vb[e, pl.ds(c * _L, _L)] * exr
            sk, sv = plsc.sort_key_val(dstv, ex)
            for s in (1, 2, 4, 8):
                kprev = _take(sk, jnp.maximum(iota - s, 0))
                vprev = _take(sv, jnp.maximum(iota - s, 0))
                same = (kprev == sk) & (iota >= s)
                sv = jnp.where(same, sv + vprev, sv)
            nxt = _take(sk, jnp.minimum(iota + 1, _L - 1))
            last = (sk != nxt) | (iota == _L - 1)
            plsc.addupdate_scatter(dloc, [sk], sv, mask=last)
            return 0
        lax.fori_loop(0, _B // _L, group, 0)

    def chunk(c, _):
        pltpu.async_copy(src_hbm.at[wid, c], srcch, semg0)
        pltpu.async_copy(dst_hbm.at[wid, c], dstch, semg1)
        pltpu.async_copy(logits_hbm.at[wid, c], lch, sems0)
        pltpu.make_async_copy(src_hbm.at[0, 0], srcch, semg0).wait()
        pltpu.make_async_copy(dst_hbm.at[0, 0], dstch, semg1).wait()
        pltpu.make_async_copy(logits_hbm.at[0, 0], lch, sems0).wait()

        issue_g(0, vrows0, semg0)
        issue_g(1, vrows1, semg1)
        wait_rows(vrows0, semg0)
        compute(0, vrows0)
        issue_s(0, vrows0, sems0)

        def pair(k, _):
            a = 2 * k + 1
            wait_rows(vrows1, semg1)
            compute(a, vrows1)
            wait_s(vrows0, sems0)
            issue_g(a + 1, vrows0, semg0)
            issue_s(a, vrows1, sems1)

            wait_rows(vrows0, semg0)
            compute(a + 1, vrows0)
            wait_s(vrows1, sems1)

            @pl.when(k != (_CH - 3) // 2)
            def _():
                issue_g(a + 2, vrows1, semg1)
            issue_s(a + 1, vrows0, sems0)
            return 0
        lax.fori_loop(0, (_CH - 1) // 2, pair, 0)
        wait_s(vrows0, sems0)
        return 0
    lax.fori_loop(0, _NCHUNK, chunk, 0)
    plsc.subcore_barrier()

    pltpu.sync_copy(shared_agg.at[pl.ds(sid * _SLICE, _SLICE), :],
                    aggp_out.at[cid, pl.ds(sid * _SLICE, _SLICE), :])

    pltpu.sync_copy(dloc, dstage_out.at[cid, sid])
    plsc.subcore_barrier()
    soff = sid * _SLICE
    def rzero(i, _):
        redacc[pl.ds(i * _L, _L)] = jnp.zeros((_L,), jnp.float32)
        return 0
    lax.fori_loop(0, _SLICE // _L, rzero, 0)
    for w in range(_NS // 4):
        for j in range(4):
            pltpu.async_copy(dstage_out.at[cid, 4 * w + j,
                                           pl.ds(soff, _SLICE)],
                             redbufs4.at[j], semg0)
        for j in range(4):
            pltpu.make_async_copy(dstage_out.at[cid, 0, pl.ds(soff, _SLICE)],
                                  redbufs4.at[0], semg0).wait()

        def redbody(i, _):
            v = redbufs4[0, pl.ds(i * _L, _L)]
            for j in range(1, 4):
                v = v + redbufs4[j, pl.ds(i * _L, _L)]
            redacc[pl.ds(i * _L, _L)] = redacc[pl.ds(i * _L, _L)] + v
            return 0
        lax.fori_loop(0, _SLICE // _L, redbody, 0)
    pltpu.sync_copy(redacc, dsc_out.at[cid, pl.ds(soff, _SLICE)])


def _proj_body(x_ref, w_ref, b_ref, o_ref):
    o_ref[...] = (jnp.dot(x_ref[...], w_ref[...],
                          preferred_element_type=jnp.float32) + b_ref[...])


def _proj(x, W, b):
    blk = 1000
    return pl.pallas_call(
        _proj_body,
        grid=(_N // blk,),
        in_specs=[pl.BlockSpec((blk, W.shape[0]), lambda i: (i, 0)),
                  pl.BlockSpec(W.shape, lambda i: (0, 0)),
                  pl.BlockSpec((1, W.shape[1]), lambda i: (0, 0))],
        out_specs=pl.BlockSpec((blk, W.shape[1]), lambda i: (i, 0)),
        out_shape=jax.ShapeDtypeStruct((_N, W.shape[1]), jnp.float32),
    )(x, W, b.reshape(1, -1))


def _comb_body(p0_ref, p1_ref, d0_ref, d1_ref, s_ref, w_ref, b_ref, o_ref):
    h = ((p0_ref[...] + p1_ref[...]) /
         (d0_ref[...] + d1_ref[...] + 1e-16) + s_ref[...])
    h = jnp.maximum(h, 0.0)
    o_ref[...] = (jnp.dot(h, w_ref[...],
                          preferred_element_type=jnp.float32) + b_ref[...])


def _comb_proj(p0, p1, d0, d1, skip, W, b):
    blk = 1000
    return pl.pallas_call(
        _comb_body,
        grid=(_N // blk,),
        in_specs=[pl.BlockSpec((blk, _D), lambda i: (i, 0)),
                  pl.BlockSpec((blk, _D), lambda i: (i, 0)),
                  pl.BlockSpec((blk, 1), lambda i: (i, 0)),
                  pl.BlockSpec((blk, 1), lambda i: (i, 0)),
                  pl.BlockSpec((blk, _D), lambda i: (i, 0)),
                  pl.BlockSpec(W.shape, lambda i: (0, 0)),
                  pl.BlockSpec((1, W.shape[1]), lambda i: (0, 0))],
        out_specs=pl.BlockSpec((blk, W.shape[1]), lambda i: (i, 0)),
        out_shape=jax.ShapeDtypeStruct((_N, W.shape[1]), jnp.float32),
    )(p0, p1, d0, d1, skip, W, b.reshape(1, -1))


def _pool_body(p0_ref, p1_ref, d0_ref, d1_ref, s_ref, batch_ref, o_ref,
               cnt_ref):
    i = pl.program_id(0)

    @pl.when(i == 0)
    def _():
        o_ref[...] = jnp.zeros_like(o_ref)
        cnt_ref[...] = jnp.zeros_like(cnt_ref)

    h = ((p0_ref[...] + p1_ref[...]) /
         (d0_ref[...] + d1_ref[...] + 1e-16) + s_ref[...])
    row = batch_ref[...].reshape(1, -1)
    gid = lax.broadcasted_iota(jnp.int32, (_G, row.shape[1]), 0)
    oh = (gid == row).astype(jnp.float32)
    o_ref[...] += jnp.dot(oh, h, preferred_element_type=jnp.float32)
    cnt_ref[...] += jnp.sum(oh, axis=1, keepdims=True)

    @pl.when(i == pl.num_programs(0) - 1)
    def _():
        o_ref[...] = o_ref[...] / jnp.maximum(cnt_ref[...], 1.0)


def _pool(p0, p1, d0, d1, skip, batch2d):
    blk = 1000
    return pl.pallas_call(
        _pool_body,
        grid=(_N // blk,),
        in_specs=[pl.BlockSpec((blk, _D), lambda i: (i, 0)),
                  pl.BlockSpec((blk, _D), lambda i: (i, 0)),
                  pl.BlockSpec((blk, 1), lambda i: (i, 0)),
                  pl.BlockSpec((blk, 1), lambda i: (i, 0)),
                  pl.BlockSpec((blk, _D), lambda i: (i, 0)),
                  pl.BlockSpec((1, 1, blk), lambda i: (i, 0, 0))],
        out_specs=pl.BlockSpec((_G, _D), lambda i: (0, 0)),
        out_shape=jax.ShapeDtypeStruct((_G, _D), jnp.float32),
        scratch_shapes=[pltpu.VMEM((_G, _D), jnp.float32)],
    )(p0, p1, d0, d1, skip, batch2d)


def kernel(x, edge_index, edge_attr, batch, Wq1, bq1, Wk1, bk1, Wv1, bv1,
           Ws1, bs1, Wq2, bq2, Wk2, bk2, Wv2, bv2, Ws2, bs2):
    x = x.astype(jnp.float32)
    src = edge_index[0].reshape(_NW, _NBLK, _B)
    dst = edge_index[1].reshape(_NW, _NBLK, _B)

    W1 = jnp.concatenate([Wq1, Wk1, Wv1, Ws1], axis=1)
    b1 = jnp.concatenate([bq1, bk1, bv1, bs1])
    p1 = _proj(x, W1, b1)
    q1, k1, v1, s1 = (p1[:, :_D], p1[:, _D:2 * _D],
                      p1[:, 2 * _D:3 * _D], p1[:, 3 * _D:])

    logits1, m1 = _sc_logits_max(q1, k1, src, dst)
    src4 = src.reshape(_NW, _NCHUNK, _CH, _B)
    dst4 = dst.reshape(_NW, _NCHUNK, _CH, _B)
    ldiff1 = _sc_subm(logits1, dst, m1)
    aggp1, _dstage1, d1 = _sc_agg(
        v1, src4, dst4, ldiff1.reshape(_NW, _NCHUNK, _CH, _B))

    W2 = jnp.concatenate([Wq2, Wk2, Wv2, Ws2], axis=1)
    b2 = jnp.concatenate([bq2, bk2, bv2, bs2])
    p2 = _comb_proj(aggp1[0, :_N], aggp1[1, :_N],
                    d1[0, :_N, None], d1[1, :_N, None], s1, W2, b2)
    q2, k2, v2, s2 = (p2[:, :_D], p2[:, _D:2 * _D],
                      p2[:, 2 * _D:3 * _D], p2[:, 3 * _D:])

    logits2, m2 = _sc_logits_max(q2, k2, src, dst)
    ldiff2 = _sc_subm(logits2, dst, m2)
    aggp2, _dstage2, d2 = _sc_agg(
        v2, src4, dst4, ldiff2.reshape(_NW, _NCHUNK, _CH, _B))

    return _pool(aggp2[0, :_N], aggp2[1, :_N],
                 d2[0, :_N, None], d2[1, :_N, None], s2,
                 batch.reshape(10, 1, _N // 10))
